# Initial kernel scaffold; baseline (speedup 1.0000x reference)
import jax
import jax.numpy as jnp
from jax.experimental import pallas as pl


def kernel(x, adj, W1, b1, W1_1, b1_1, W2, b2, W2_1, b2_1):
    raise NotImplementedError("write your pallas kernel here")


# speedup vs baseline: 1.8638x; 1.8638x over previous
import functools

import jax
import jax.numpy as jnp
from jax.experimental import pallas as pl
from jax.experimental.pallas import tpu as pltpu


def _pick_block(n, target=512):
    for bm in (512, 400, 256, 200, 128, 80, 64, 40, 32, 16, 8):
        if bm <= target and n % bm == 0:
            return bm
    return n


def _pass1_body(x_ref, wc_ref, bc_ref, adj_ref, h_ref, s_ref, *, hdim):
    i = pl.program_id(0)

    @pl.when(i == 0)
    def _():
        s_ref[...] = jnp.dot(x_ref[...], wc_ref[...],
                             preferred_element_type=jnp.float32)

    y = jnp.dot(adj_ref[...], s_ref[...],
                preferred_element_type=jnp.float32) + bc_ref[...]
    y = jnp.maximum(y, 0.0)
    h_ref[...] = y[:, :hdim] * y[:, hdim:]


def _pass2_body(h_ref, wc_ref, bc_ref, adj_ref, o_ref, t_ref, *, cdim):
    i = pl.program_id(0)

    @pl.when(i == 0)
    def _():
        t_ref[...] = jnp.dot(h_ref[...], wc_ref[...],
                             preferred_element_type=jnp.float32)

    y = jnp.dot(adj_ref[...], t_ref[...],
                preferred_element_type=jnp.float32) + bc_ref[...]
    v = y[:, :cdim] * y[:, cdim:]
    m = jnp.max(v, axis=1, keepdims=True)
    e = jnp.exp(v - m)
    o_ref[...] = (v - m) - jnp.log(jnp.sum(e, axis=1, keepdims=True))


def kernel(x, adj, W1, b1, W1_1, b1_1, W2, b2, W2_1, b2_1):
    n, nfeat = x.shape
    nhid = W1.shape[1]
    nclass = W2.shape[1]
    bm = _pick_block(n)
    grid = (n // bm,)

    wc1 = jnp.concatenate([W1, W1_1], axis=1)
    bc1 = jnp.concatenate([b1, b1_1])[None, :]
    wc2 = jnp.concatenate([W2, W2_1], axis=1)
    bc2 = jnp.concatenate([b2, b2_1])[None, :]

    h = pl.pallas_call(
        functools.partial(_pass1_body, hdim=nhid),
        grid=grid,
        in_specs=[
            pl.BlockSpec((n, nfeat), lambda i: (0, 0)),
            pl.BlockSpec((nfeat, 2 * nhid), lambda i: (0, 0)),
            pl.BlockSpec((1, 2 * nhid), lambda i: (0, 0)),
            pl.BlockSpec((bm, n), lambda i: (i, 0)),
        ],
        out_specs=pl.BlockSpec((bm, nhid), lambda i: (i, 0)),
        out_shape=jax.ShapeDtypeStruct((n, nhid), jnp.float32),
        scratch_shapes=[pltpu.VMEM((n, 2 * nhid), jnp.float32)],
        compiler_params=pltpu.CompilerParams(
            dimension_semantics=("arbitrary",)),
    )(x, wc1, bc1, adj)

    out = pl.pallas_call(
        functools.partial(_pass2_body, cdim=nclass),
        grid=grid,
        in_specs=[
            pl.BlockSpec((n, nhid), lambda i: (0, 0)),
            pl.BlockSpec((nhid, 2 * nclass), lambda i: (0, 0)),
            pl.BlockSpec((1, 2 * nclass), lambda i: (0, 0)),
            pl.BlockSpec((bm, n), lambda i: (i, 0)),
        ],
        out_specs=pl.BlockSpec((bm, nclass), lambda i: (i, 0)),
        out_shape=jax.ShapeDtypeStruct((n, nclass), jnp.float32),
        scratch_shapes=[pltpu.VMEM((n, 2 * nclass), jnp.float32)],
        compiler_params=pltpu.CompilerParams(
            dimension_semantics=("arbitrary",)),
    )(h, wc2, bc2, adj)

    return out


# speedup vs baseline: 1.9228x; 1.0317x over previous
import functools

import jax
import jax.numpy as jnp
from jax.experimental import pallas as pl
from jax.experimental.pallas import tpu as pltpu


def _pick_block(n, target=512):
    for bm in (512, 400, 256, 200, 128, 80, 64, 40, 32, 16, 8):
        if bm <= target and n % bm == 0:
            return bm
    return n


def _body(x_ref, wc1_ref, bc1_ref, wc2_ref, bc2_ref, adj_ref, o_ref,
          s_ref, t_ref, h_ref, *, bm, hdim, cdim):
    p = pl.program_id(0)
    j = pl.program_id(1)

    @pl.when(jnp.logical_and(p == 0, j == 0))
    def _():
        s_ref[...] = jnp.dot(x_ref[...], wc1_ref[...],
                             preferred_element_type=jnp.float32)

    @pl.when(p == 0)
    def _():
        y = jnp.dot(adj_ref[...], s_ref[...],
                    preferred_element_type=jnp.float32) + bc1_ref[...]
        y = jnp.maximum(y, 0.0)
        h_ref[pl.ds(j * bm, bm), :] = y[:, :hdim] * y[:, hdim:]

    @pl.when(jnp.logical_and(p == 1, j == 0))
    def _():
        t_ref[...] = jnp.dot(h_ref[...], wc2_ref[...],
                             preferred_element_type=jnp.float32)

    @pl.when(p == 1)
    def _():
        y = jnp.dot(adj_ref[...], t_ref[...],
                    preferred_element_type=jnp.float32) + bc2_ref[...]
        v = y[:, :cdim] * y[:, cdim:]
        m = jnp.max(v, axis=1, keepdims=True)
        e = jnp.exp(v - m)
        o_ref[...] = (v - m) - jnp.log(jnp.sum(e, axis=1, keepdims=True))


def kernel(x, adj, W1, b1, W1_1, b1_1, W2, b2, W2_1, b2_1):
    n, nfeat = x.shape
    nhid = W1.shape[1]
    nclass = W2.shape[1]
    bm = _pick_block(n)

    wc1 = jnp.concatenate([W1, W1_1], axis=1)
    bc1 = jnp.concatenate([b1, b1_1])[None, :]
    wc2 = jnp.concatenate([W2, W2_1], axis=1)
    bc2 = jnp.concatenate([b2, b2_1])[None, :]

    out = pl.pallas_call(
        functools.partial(_body, bm=bm, hdim=nhid, cdim=nclass),
        grid=(2, n // bm),
        in_specs=[
            pl.BlockSpec((n, nfeat), lambda p, j: (0, 0)),
            pl.BlockSpec((nfeat, 2 * nhid), lambda p, j: (0, 0)),
            pl.BlockSpec((1, 2 * nhid), lambda p, j: (0, 0)),
            pl.BlockSpec((nhid, 2 * nclass), lambda p, j: (0, 0)),
            pl.BlockSpec((1, 2 * nclass), lambda p, j: (0, 0)),
            pl.BlockSpec((bm, n), lambda p, j: (j, 0)),
        ],
        out_specs=pl.BlockSpec((bm, nclass), lambda p, j: (j, 0)),
        out_shape=jax.ShapeDtypeStruct((n, nclass), jnp.float32),
        scratch_shapes=[
            pltpu.VMEM((n, 2 * nhid), jnp.float32),
            pltpu.VMEM((n, 2 * nclass), jnp.float32),
            pltpu.VMEM((n, nhid), jnp.float32),
        ],
        compiler_params=pltpu.CompilerParams(
            dimension_semantics=("arbitrary", "arbitrary")),
    )(x, wc1, bc1, wc2, bc2, adj)

    return out
